# baseline (speedup 1.0000x reference)
import functools

import jax
import jax.numpy as jnp
from jax import lax
from jax.experimental import pallas as pl
from jax.experimental.pallas import tpu as pltpu
from jax.experimental.pallas import tpu_sc as plsc

N = 10000
L = 64
NUM_WORKERS = 32
B = 320
NP = NUM_WORKERS * B
NWAVE = B // 16
PW = 176
UW = 512
JBLK = 4
THR = 48
RING = 4
DEPTH = 3

_SC_PARAMS = pltpu.CompilerParams(needs_layout_passes=False)
_MESH = plsc.VectorSubcoreMesh(core_axis_name="c", subcore_axis_name="s")


def _sc_parent_labels(parents, true_labels):
    @functools.partial(
        pl.kernel,
        out_type=jax.ShapeDtypeStruct((NP,), jnp.int32),
        mesh=_MESH,
        scratch_types=[
            pltpu.VMEM((B,), jnp.int32),
            pltpu.VMEM((PW,), jnp.int32),
            pltpu.VMEM((B,), jnp.int32),
        ],
        compiler_params=_SC_PARAMS,
    )
    def k(parents_hbm, labels_hbm, plbl_hbm, par_v, labw_v, plbl_v):
        wid = lax.axis_index("s") * 2 + lax.axis_index("c")
        base = wid * B
        w0 = pl.multiple_of(jnp.maximum(base // 2 - 8, 0), 8)
        pltpu.sync_copy(parents_hbm.at[pl.ds(base, B)], par_v)
        pltpu.sync_copy(labels_hbm.at[pl.ds(w0, PW)], labw_v)
        for w in range(NWAVE):
            s = pl.ds(w * 16, 16)
            idx = jnp.clip(par_v[s] - w0, 0, PW - 1)
            plbl_v[s] = plsc.load_gather(labw_v, [idx])
        pltpu.sync_copy(plbl_v, plbl_hbm.at[pl.ds(base, B)])

    return k(parents, true_labels)


def _sc_unary_edge_partials(true_labels, unary_t, edge_t, plbl_pad):
    @functools.partial(
        pl.kernel,
        out_type=jax.ShapeDtypeStruct((NUM_WORKERS, 16), jnp.float32),
        mesh=_MESH,
        scratch_types=[
            pltpu.VMEM((B,), jnp.int32),
            pltpu.VMEM((B,), jnp.int32),
            pltpu.VMEM((L, UW), jnp.float32),
            pltpu.VMEM((RING * 16, 8, 128), jnp.float32),
            pltpu.VMEM((16,), jnp.float32),
            pltpu.SemaphoreType.DMA,
            pltpu.SemaphoreType.DMA,
        ],
        compiler_params=_SC_PARAMS,
    )
    def k(labels_hbm, unary_hbm, edge_hbm, plbl_hbm, out_hbm,
          lbl_v, pk_v, ublk, ering, acc_v, sem_u, sem_w):
        wid = lax.axis_index("s") * 2 + lax.axis_index("c")
        base = wid * B
        lane = lax.iota(jnp.int32, 16)
        pltpu.sync_copy(labels_hbm.at[pl.ds(base, B)], lbl_v)
        pltpu.sync_copy(plbl_hbm.at[pl.ds(base, B)], pk_v)
        ubase = pl.multiple_of(
            jnp.minimum((base // 128) * 128, 9600), 128)
        ucopy = pltpu.async_copy(
            unary_hbm.at[pl.ds(0, L), pl.ds(ubase, UW)], ublk, sem_u)

        for w in range(NWAVE):
            s = pl.ds(w * 16, 16)
            node = base + w * 16 + lane
            pk_v[s] = jnp.where(node < N, pk_v[s] * L + lbl_v[s], 0)

        def wave_i0(w):
            return jnp.minimum(((base + w * 16) // 128) * 128, 9984)

        def fire(w):
            pv = pk_v[pl.ds(w * 16, 16)]
            slot = jnp.bitwise_and(w, RING - 1) * 16
            i0 = pl.multiple_of(wave_i0(w), 128)
            for jj in range(16):
                p = jnp.sum(jnp.where(lane == jj, pv, 0))

                @pl.when(p >= THR * L)
                def _():
                    j = jnp.right_shift(p, 6)
                    l0 = pl.multiple_of(jnp.bitwise_and(p, 56), 8)
                    pltpu.async_copy(
                        edge_hbm.at[j, pl.ds(l0, 8), pl.ds(i0, 128)],
                        ering.at[slot + jj], sem_w)

        def drain(w):
            pv = pk_v[pl.ds(w * 16, 16)]
            c = jnp.sum(jnp.where(pv >= THR * L, 1, 0))

            def body(_, carry):
                pltpu.make_async_copy(
                    edge_hbm.at[pl.ds(0, 1), pl.ds(0, 8), pl.ds(0, 128)],
                    ering.at[pl.ds(0, 1)], sem_w).wait()
                return carry

            lax.fori_loop(0, c, body, jnp.int32(0))

        def select(w, acc):
            s = pl.ds(w * 16, 16)
            node = base + w * 16 + lane
            slot = jnp.bitwise_and(w, RING - 1) * 16
            pv = pk_v[s]
            lbl = lbl_v[s]
            valid = node < N
            emask = valid & (node >= 1) & (pv >= THR * L)
            ve = plsc.load_gather(
                ering, [slot + lane, jnp.bitwise_and(lbl, 7),
                        jnp.where(valid, node - wave_i0(w), 0)])
            vu = plsc.load_gather(
                ublk, [jnp.bitwise_and(lbl, L - 1),
                       jnp.where(valid, node - ubase, 0)])
            acc = acc + jnp.where(valid, vu, 0.0)
            return acc + jnp.where(emask, ve, 0.0)

        ucopy.wait()
        for w in range(DEPTH):
            fire(jnp.int32(w))

        def wave_body(w, acc):
            pl.when(w + DEPTH < NWAVE)(lambda: fire(w + DEPTH))
            drain(w)
            return select(w, acc)

        acc = lax.fori_loop(0, NWAVE, wave_body,
                            jnp.zeros((16,), jnp.float32))
        acc_v[...] = acc
        pltpu.sync_copy(acc_v, out_hbm.at[wid])

    return k(true_labels, unary_t, edge_t, plbl_pad)


def _tc_edge_body(plbl_ref, lbl_ref, bel_ref, edge_ref, out_ref):
    g = pl.program_id(0)
    kk = lax.broadcasted_iota(jnp.int32, (1, L, N), 1)
    ii = lax.broadcasted_iota(jnp.int32, (1, L, N), 2)
    lblb = lbl_ref[...].reshape(1, 1, N)
    plblb = plbl_ref[...].reshape(1, 1, N)
    s = jnp.float32(0.0)
    for dj in range(JBLK):
        sel = ((plblb == g * JBLK + dj) & (kk == lblb) & (ii >= 1))
        s = s + jnp.sum(jnp.where(sel, edge_ref[pl.ds(dj, 1)], 0.0))

    @pl.when(g == 0)
    def _():
        bel = bel_ref[...]
        m = jnp.max(bel)
        z = m + jnp.log(jnp.sum(jnp.exp(bel - m)))
        out_ref[...] = jnp.broadcast_to(z, (1, 1))

    out_ref[...] = out_ref[...] - s


def _tc_final_body(zme_ref, part_ref, out_ref):
    out_ref[...] = zme_ref[...] - jnp.sum(part_ref[...])


def kernel(unary_potentials, edge_potentials, beliefs, parents, true_labels):
    parents = parents.astype(jnp.int32)
    labels = true_labels.astype(jnp.int32)
    unary_t = jnp.transpose(unary_potentials, (1, 0))
    edge_t = jnp.transpose(edge_potentials, (1, 2, 0))

    plbl_pad = _sc_parent_labels(parents, labels)
    partials = _sc_unary_edge_partials(labels, unary_t, edge_t, plbl_pad)
    lbl2d = labels.reshape(1, N)

    zme = pl.pallas_call(
        _tc_edge_body,
        grid=(THR // JBLK,),
        in_specs=[
            pl.BlockSpec((1, N), lambda g: (0, 0)),
            pl.BlockSpec((1, N), lambda g: (0, 0)),
            pl.BlockSpec((1, L), lambda g: (0, 0)),
            pl.BlockSpec((JBLK, L, N), lambda g: (g, 0, 0)),
        ],
        out_specs=pl.BlockSpec((1, 1), lambda g: (0, 0)),
        out_shape=jax.ShapeDtypeStruct((1, 1), jnp.float32),
    )(plbl_pad[:N].reshape(1, N), lbl2d, beliefs[0:1, :], edge_t)

    out = pl.pallas_call(
        _tc_final_body,
        out_shape=jax.ShapeDtypeStruct((1, 1), jnp.float32),
    )(zme, partials)
    return out[0, 0]


# speedup vs baseline: 1.0673x; 1.0673x over previous
import functools

import jax
import jax.numpy as jnp
from jax import lax
from jax.experimental import pallas as pl
from jax.experimental.pallas import tpu as pltpu
from jax.experimental.pallas import tpu_sc as plsc

N = 10000
L = 64
NUM_WORKERS = 32
B = 320
NP = NUM_WORKERS * B
NWAVE = B // 16
PW = 176
UW = 512
JBLK = 4
THR = 40
RING = 4
DEPTH = 3

_SC_PARAMS = pltpu.CompilerParams(needs_layout_passes=False)
_MESH = plsc.VectorSubcoreMesh(core_axis_name="c", subcore_axis_name="s")


def _sc_parent_labels(parents, true_labels):
    @functools.partial(
        pl.kernel,
        out_type=jax.ShapeDtypeStruct((NP,), jnp.int32),
        mesh=_MESH,
        scratch_types=[
            pltpu.VMEM((B,), jnp.int32),
            pltpu.VMEM((PW,), jnp.int32),
            pltpu.VMEM((B,), jnp.int32),
        ],
        compiler_params=_SC_PARAMS,
    )
    def k(parents_hbm, labels_hbm, plbl_hbm, par_v, labw_v, plbl_v):
        wid = lax.axis_index("s") * 2 + lax.axis_index("c")
        base = wid * B
        w0 = pl.multiple_of(jnp.maximum(base // 2 - 8, 0), 8)
        pltpu.sync_copy(parents_hbm.at[pl.ds(base, B)], par_v)
        pltpu.sync_copy(labels_hbm.at[pl.ds(w0, PW)], labw_v)
        for w in range(NWAVE):
            s = pl.ds(w * 16, 16)
            idx = jnp.clip(par_v[s] - w0, 0, PW - 1)
            plbl_v[s] = plsc.load_gather(labw_v, [idx])
        pltpu.sync_copy(plbl_v, plbl_hbm.at[pl.ds(base, B)])

    return k(parents, true_labels)


def _sc_unary_edge_partials(true_labels, unary_t, edge_t, plbl_pad):
    @functools.partial(
        pl.kernel,
        out_type=jax.ShapeDtypeStruct((NUM_WORKERS, 16), jnp.float32),
        mesh=_MESH,
        scratch_types=[
            pltpu.VMEM((B,), jnp.int32),
            pltpu.VMEM((B,), jnp.int32),
            pltpu.VMEM((L, UW), jnp.float32),
            pltpu.VMEM((RING * 16, 8, 128), jnp.float32),
            pltpu.VMEM((16,), jnp.float32),
            pltpu.SemaphoreType.DMA,
            pltpu.SemaphoreType.DMA,
        ],
        compiler_params=_SC_PARAMS,
    )
    def k(labels_hbm, unary_hbm, edge_hbm, plbl_hbm, out_hbm,
          lbl_v, pk_v, ublk, ering, acc_v, sem_u, sem_w):
        wid = lax.axis_index("s") * 2 + lax.axis_index("c")
        base = wid * B
        lane = lax.iota(jnp.int32, 16)
        pltpu.sync_copy(labels_hbm.at[pl.ds(base, B)], lbl_v)
        pltpu.sync_copy(plbl_hbm.at[pl.ds(base, B)], pk_v)
        ubase = pl.multiple_of(
            jnp.minimum((base // 128) * 128, 9600), 128)
        ucopy = pltpu.async_copy(
            unary_hbm.at[pl.ds(0, L), pl.ds(ubase, UW)], ublk, sem_u)

        for w in range(NWAVE):
            s = pl.ds(w * 16, 16)
            node = base + w * 16 + lane
            pk_v[s] = jnp.where(node < N, pk_v[s] * L + lbl_v[s], 0)

        def wave_i0(w):
            return jnp.minimum(((base + w * 16) // 128) * 128, 9984)

        def fire(w):
            pv = pk_v[pl.ds(w * 16, 16)]
            slot = jnp.bitwise_and(w, RING - 1) * 16
            i0 = pl.multiple_of(wave_i0(w), 128)
            for jj in range(16):
                p = jnp.sum(jnp.where(lane == jj, pv, 0))

                @pl.when(p >= THR * L)
                def _():
                    j = jnp.right_shift(p, 6)
                    l0 = pl.multiple_of(jnp.bitwise_and(p, 56), 8)
                    pltpu.async_copy(
                        edge_hbm.at[j, pl.ds(l0, 8), pl.ds(i0, 128)],
                        ering.at[slot + jj], sem_w)

        def drain(w):
            pv = pk_v[pl.ds(w * 16, 16)]
            c = jnp.sum(jnp.where(pv >= THR * L, 1, 0))

            def body(_, carry):
                pltpu.make_async_copy(
                    edge_hbm.at[pl.ds(0, 1), pl.ds(0, 8), pl.ds(0, 128)],
                    ering.at[pl.ds(0, 1)], sem_w).wait()
                return carry

            lax.fori_loop(0, c, body, jnp.int32(0))

        def select(w, acc):
            s = pl.ds(w * 16, 16)
            node = base + w * 16 + lane
            slot = jnp.bitwise_and(w, RING - 1) * 16
            pv = pk_v[s]
            lbl = lbl_v[s]
            valid = node < N
            emask = valid & (node >= 1) & (pv >= THR * L)
            ve = plsc.load_gather(
                ering, [slot + lane, jnp.bitwise_and(lbl, 7),
                        jnp.where(valid, node - wave_i0(w), 0)])
            vu = plsc.load_gather(
                ublk, [jnp.bitwise_and(lbl, L - 1),
                       jnp.where(valid, node - ubase, 0)])
            acc = acc + jnp.where(valid, vu, 0.0)
            return acc + jnp.where(emask, ve, 0.0)

        ucopy.wait()
        for w in range(DEPTH):
            fire(jnp.int32(w))

        def wave_body(w, acc):
            pl.when(w + DEPTH < NWAVE)(lambda: fire(w + DEPTH))
            drain(w)
            return select(w, acc)

        acc = lax.fori_loop(0, NWAVE, wave_body,
                            jnp.zeros((16,), jnp.float32))
        acc_v[...] = acc
        pltpu.sync_copy(acc_v, out_hbm.at[wid])

    return k(true_labels, unary_t, edge_t, plbl_pad)


def _tc_edge_body(plbl_ref, lbl_ref, bel_ref, edge_ref, out_ref):
    g = pl.program_id(0)
    kk = lax.broadcasted_iota(jnp.int32, (1, L, N), 1)
    ii = lax.broadcasted_iota(jnp.int32, (1, L, N), 2)
    lblb = lbl_ref[...].reshape(1, 1, N)
    plblb = plbl_ref[...].reshape(1, 1, N)
    s = jnp.float32(0.0)
    for dj in range(JBLK):
        sel = ((plblb == g * JBLK + dj) & (kk == lblb) & (ii >= 1))
        s = s + jnp.sum(jnp.where(sel, edge_ref[pl.ds(dj, 1)], 0.0))

    @pl.when(g == 0)
    def _():
        bel = bel_ref[...]
        m = jnp.max(bel)
        z = m + jnp.log(jnp.sum(jnp.exp(bel - m)))
        out_ref[...] = jnp.broadcast_to(z, (1, 1))

    out_ref[...] = out_ref[...] - s


def _tc_final_body(zme_ref, part_ref, out_ref):
    out_ref[...] = zme_ref[...] - jnp.sum(part_ref[...])


def kernel(unary_potentials, edge_potentials, beliefs, parents, true_labels):
    parents = parents.astype(jnp.int32)
    labels = true_labels.astype(jnp.int32)
    unary_t = jnp.transpose(unary_potentials, (1, 0))
    edge_t = jnp.transpose(edge_potentials, (1, 2, 0))

    plbl_pad = _sc_parent_labels(parents, labels)
    partials = _sc_unary_edge_partials(labels, unary_t, edge_t, plbl_pad)
    lbl2d = labels.reshape(1, N)

    zme = pl.pallas_call(
        _tc_edge_body,
        grid=(THR // JBLK,),
        in_specs=[
            pl.BlockSpec((1, N), lambda g: (0, 0)),
            pl.BlockSpec((1, N), lambda g: (0, 0)),
            pl.BlockSpec((1, L), lambda g: (0, 0)),
            pl.BlockSpec((JBLK, L, N), lambda g: (g, 0, 0)),
        ],
        out_specs=pl.BlockSpec((1, 1), lambda g: (0, 0)),
        out_shape=jax.ShapeDtypeStruct((1, 1), jnp.float32),
    )(plbl_pad[:N].reshape(1, N), lbl2d, beliefs[0:1, :], edge_t)

    out = pl.pallas_call(
        _tc_final_body,
        out_shape=jax.ShapeDtypeStruct((1, 1), jnp.float32),
    )(zme, partials)
    return out[0, 0]


# speedup vs baseline: 1.1060x; 1.0362x over previous
import functools

import jax
import jax.numpy as jnp
from jax import lax
from jax.experimental import pallas as pl
from jax.experimental.pallas import tpu as pltpu
from jax.experimental.pallas import tpu_sc as plsc

N = 10000
L = 64
NUM_WORKERS = 32
B = 320
NP = NUM_WORKERS * B
NWAVE = B // 16
PW = 176
UW = 512
JBLK = 4
THR = 36
RING = 4
DEPTH = 3

_SC_PARAMS = pltpu.CompilerParams(needs_layout_passes=False)
_MESH = plsc.VectorSubcoreMesh(core_axis_name="c", subcore_axis_name="s")


def _sc_parent_labels(parents, true_labels):
    @functools.partial(
        pl.kernel,
        out_type=jax.ShapeDtypeStruct((NP,), jnp.int32),
        mesh=_MESH,
        scratch_types=[
            pltpu.VMEM((B,), jnp.int32),
            pltpu.VMEM((PW,), jnp.int32),
            pltpu.VMEM((B,), jnp.int32),
        ],
        compiler_params=_SC_PARAMS,
    )
    def k(parents_hbm, labels_hbm, plbl_hbm, par_v, labw_v, plbl_v):
        wid = lax.axis_index("s") * 2 + lax.axis_index("c")
        base = wid * B
        w0 = pl.multiple_of(jnp.maximum(base // 2 - 8, 0), 8)
        pltpu.sync_copy(parents_hbm.at[pl.ds(base, B)], par_v)
        pltpu.sync_copy(labels_hbm.at[pl.ds(w0, PW)], labw_v)
        for w in range(NWAVE):
            s = pl.ds(w * 16, 16)
            idx = jnp.clip(par_v[s] - w0, 0, PW - 1)
            plbl_v[s] = plsc.load_gather(labw_v, [idx])
        pltpu.sync_copy(plbl_v, plbl_hbm.at[pl.ds(base, B)])

    return k(parents, true_labels)


def _sc_unary_edge_partials(true_labels, unary_t, edge_t, plbl_pad):
    @functools.partial(
        pl.kernel,
        out_type=jax.ShapeDtypeStruct((NUM_WORKERS, 16), jnp.float32),
        mesh=_MESH,
        scratch_types=[
            pltpu.VMEM((B,), jnp.int32),
            pltpu.VMEM((B,), jnp.int32),
            pltpu.VMEM((L, UW), jnp.float32),
            pltpu.VMEM((RING * 16, 8, 128), jnp.float32),
            pltpu.VMEM((16,), jnp.float32),
            pltpu.SemaphoreType.DMA,
            pltpu.SemaphoreType.DMA,
        ],
        compiler_params=_SC_PARAMS,
    )
    def k(labels_hbm, unary_hbm, edge_hbm, plbl_hbm, out_hbm,
          lbl_v, pk_v, ublk, ering, acc_v, sem_u, sem_w):
        wid = lax.axis_index("s") * 2 + lax.axis_index("c")
        base = wid * B
        lane = lax.iota(jnp.int32, 16)
        pltpu.sync_copy(labels_hbm.at[pl.ds(base, B)], lbl_v)
        pltpu.sync_copy(plbl_hbm.at[pl.ds(base, B)], pk_v)
        ubase = pl.multiple_of(
            jnp.minimum((base // 128) * 128, 9600), 128)
        ucopy = pltpu.async_copy(
            unary_hbm.at[pl.ds(0, L), pl.ds(ubase, UW)], ublk, sem_u)

        for w in range(NWAVE):
            s = pl.ds(w * 16, 16)
            node = base + w * 16 + lane
            pk_v[s] = jnp.where(node < N, pk_v[s] * L + lbl_v[s], 0)

        def wave_i0(w):
            return jnp.minimum(((base + w * 16) // 128) * 128, 9984)

        def fire(w):
            pv = pk_v[pl.ds(w * 16, 16)]
            slot = jnp.bitwise_and(w, RING - 1) * 16
            i0 = pl.multiple_of(wave_i0(w), 128)
            for jj in range(16):
                p = jnp.sum(jnp.where(lane == jj, pv, 0))

                @pl.when(p >= THR * L)
                def _():
                    j = jnp.right_shift(p, 6)
                    l0 = pl.multiple_of(jnp.bitwise_and(p, 56), 8)
                    pltpu.async_copy(
                        edge_hbm.at[j, pl.ds(l0, 8), pl.ds(i0, 128)],
                        ering.at[slot + jj], sem_w)

        def drain(w):
            pv = pk_v[pl.ds(w * 16, 16)]
            c = jnp.sum(jnp.where(pv >= THR * L, 1, 0))

            def body(_, carry):
                pltpu.make_async_copy(
                    edge_hbm.at[pl.ds(0, 1), pl.ds(0, 8), pl.ds(0, 128)],
                    ering.at[pl.ds(0, 1)], sem_w).wait()
                return carry

            lax.fori_loop(0, c, body, jnp.int32(0))

        def select(w, acc):
            s = pl.ds(w * 16, 16)
            node = base + w * 16 + lane
            slot = jnp.bitwise_and(w, RING - 1) * 16
            pv = pk_v[s]
            lbl = lbl_v[s]
            valid = node < N
            emask = valid & (node >= 1) & (pv >= THR * L)
            ve = plsc.load_gather(
                ering, [slot + lane, jnp.bitwise_and(lbl, 7),
                        jnp.where(valid, node - wave_i0(w), 0)])
            vu = plsc.load_gather(
                ublk, [jnp.bitwise_and(lbl, L - 1),
                       jnp.where(valid, node - ubase, 0)])
            acc = acc + jnp.where(valid, vu, 0.0)
            return acc + jnp.where(emask, ve, 0.0)

        ucopy.wait()
        for w in range(DEPTH):
            fire(jnp.int32(w))

        def wave_body(w, acc):
            pl.when(w + DEPTH < NWAVE)(lambda: fire(w + DEPTH))
            drain(w)
            return select(w, acc)

        acc = lax.fori_loop(0, NWAVE, wave_body,
                            jnp.zeros((16,), jnp.float32))
        acc_v[...] = acc
        pltpu.sync_copy(acc_v, out_hbm.at[wid])

    return k(true_labels, unary_t, edge_t, plbl_pad)


def _tc_edge_body(plbl_ref, lbl_ref, bel_ref, edge_ref, out_ref):
    g = pl.program_id(0)
    kk = lax.broadcasted_iota(jnp.int32, (1, L, N), 1)
    ii = lax.broadcasted_iota(jnp.int32, (1, L, N), 2)
    lblb = lbl_ref[...].reshape(1, 1, N)
    plblb = plbl_ref[...].reshape(1, 1, N)
    s = jnp.float32(0.0)
    for dj in range(JBLK):
        sel = ((plblb == g * JBLK + dj) & (kk == lblb) & (ii >= 1))
        s = s + jnp.sum(jnp.where(sel, edge_ref[pl.ds(dj, 1)], 0.0))

    @pl.when(g == 0)
    def _():
        bel = bel_ref[...]
        m = jnp.max(bel)
        z = m + jnp.log(jnp.sum(jnp.exp(bel - m)))
        out_ref[...] = jnp.broadcast_to(z, (1, 1))

    out_ref[...] = out_ref[...] - s


def _tc_final_body(zme_ref, part_ref, out_ref):
    out_ref[...] = zme_ref[...] - jnp.sum(part_ref[...])


def kernel(unary_potentials, edge_potentials, beliefs, parents, true_labels):
    parents = parents.astype(jnp.int32)
    labels = true_labels.astype(jnp.int32)
    unary_t = jnp.transpose(unary_potentials, (1, 0))
    edge_t = jnp.transpose(edge_potentials, (1, 2, 0))

    plbl_pad = _sc_parent_labels(parents, labels)
    partials = _sc_unary_edge_partials(labels, unary_t, edge_t, plbl_pad)
    lbl2d = labels.reshape(1, N)

    zme = pl.pallas_call(
        _tc_edge_body,
        grid=(THR // JBLK,),
        in_specs=[
            pl.BlockSpec((1, N), lambda g: (0, 0)),
            pl.BlockSpec((1, N), lambda g: (0, 0)),
            pl.BlockSpec((1, L), lambda g: (0, 0)),
            pl.BlockSpec((JBLK, L, N), lambda g: (g, 0, 0)),
        ],
        out_specs=pl.BlockSpec((1, 1), lambda g: (0, 0)),
        out_shape=jax.ShapeDtypeStruct((1, 1), jnp.float32),
    )(plbl_pad[:N].reshape(1, N), lbl2d, beliefs[0:1, :], edge_t)

    out = pl.pallas_call(
        _tc_final_body,
        out_shape=jax.ShapeDtypeStruct((1, 1), jnp.float32),
    )(zme, partials)
    return out[0, 0]


# speedup vs baseline: 1.1446x; 1.0349x over previous
import functools

import jax
import jax.numpy as jnp
from jax import lax
from jax.experimental import pallas as pl
from jax.experimental.pallas import tpu as pltpu
from jax.experimental.pallas import tpu_sc as plsc

N = 10000
L = 64
NUM_WORKERS = 32
B = 320
NP = NUM_WORKERS * B
NWAVE = B // 16
PW = 176
UW = 512
JBLK = 4
THR = 32
RING = 4
DEPTH = 3

_SC_PARAMS = pltpu.CompilerParams(needs_layout_passes=False)
_MESH = plsc.VectorSubcoreMesh(core_axis_name="c", subcore_axis_name="s")


def _sc_parent_labels(parents, true_labels):
    @functools.partial(
        pl.kernel,
        out_type=jax.ShapeDtypeStruct((NP,), jnp.int32),
        mesh=_MESH,
        scratch_types=[
            pltpu.VMEM((B,), jnp.int32),
            pltpu.VMEM((PW,), jnp.int32),
            pltpu.VMEM((B,), jnp.int32),
        ],
        compiler_params=_SC_PARAMS,
    )
    def k(parents_hbm, labels_hbm, plbl_hbm, par_v, labw_v, plbl_v):
        wid = lax.axis_index("s") * 2 + lax.axis_index("c")
        base = wid * B
        w0 = pl.multiple_of(jnp.maximum(base // 2 - 8, 0), 8)
        pltpu.sync_copy(parents_hbm.at[pl.ds(base, B)], par_v)
        pltpu.sync_copy(labels_hbm.at[pl.ds(w0, PW)], labw_v)
        for w in range(NWAVE):
            s = pl.ds(w * 16, 16)
            idx = jnp.clip(par_v[s] - w0, 0, PW - 1)
            plbl_v[s] = plsc.load_gather(labw_v, [idx])
        pltpu.sync_copy(plbl_v, plbl_hbm.at[pl.ds(base, B)])

    return k(parents, true_labels)


def _sc_unary_edge_partials(true_labels, unary_t, edge_t, plbl_pad):
    @functools.partial(
        pl.kernel,
        out_type=jax.ShapeDtypeStruct((NUM_WORKERS, 16), jnp.float32),
        mesh=_MESH,
        scratch_types=[
            pltpu.VMEM((B,), jnp.int32),
            pltpu.VMEM((B,), jnp.int32),
            pltpu.VMEM((L, UW), jnp.float32),
            pltpu.VMEM((RING * 16, 8, 128), jnp.float32),
            pltpu.VMEM((16,), jnp.float32),
            pltpu.SemaphoreType.DMA,
            pltpu.SemaphoreType.DMA,
        ],
        compiler_params=_SC_PARAMS,
    )
    def k(labels_hbm, unary_hbm, edge_hbm, plbl_hbm, out_hbm,
          lbl_v, pk_v, ublk, ering, acc_v, sem_u, sem_w):
        wid = lax.axis_index("s") * 2 + lax.axis_index("c")
        base = wid * B
        lane = lax.iota(jnp.int32, 16)
        pltpu.sync_copy(labels_hbm.at[pl.ds(base, B)], lbl_v)
        pltpu.sync_copy(plbl_hbm.at[pl.ds(base, B)], pk_v)
        ubase = pl.multiple_of(
            jnp.minimum((base // 128) * 128, 9600), 128)
        ucopy = pltpu.async_copy(
            unary_hbm.at[pl.ds(0, L), pl.ds(ubase, UW)], ublk, sem_u)

        for w in range(NWAVE):
            s = pl.ds(w * 16, 16)
            node = base + w * 16 + lane
            pk_v[s] = jnp.where(node < N, pk_v[s] * L + lbl_v[s], 0)

        def wave_i0(w):
            return jnp.minimum(((base + w * 16) // 128) * 128, 9984)

        def fire(w):
            pv = pk_v[pl.ds(w * 16, 16)]
            slot = jnp.bitwise_and(w, RING - 1) * 16
            i0 = pl.multiple_of(wave_i0(w), 128)
            for jj in range(16):
                p = jnp.sum(jnp.where(lane == jj, pv, 0))

                @pl.when(p >= THR * L)
                def _():
                    j = jnp.right_shift(p, 6)
                    l0 = pl.multiple_of(jnp.bitwise_and(p, 56), 8)
                    pltpu.async_copy(
                        edge_hbm.at[j, pl.ds(l0, 8), pl.ds(i0, 128)],
                        ering.at[slot + jj], sem_w)

        def drain(w):
            pv = pk_v[pl.ds(w * 16, 16)]
            c = jnp.sum(jnp.where(pv >= THR * L, 1, 0))

            def body(_, carry):
                pltpu.make_async_copy(
                    edge_hbm.at[pl.ds(0, 1), pl.ds(0, 8), pl.ds(0, 128)],
                    ering.at[pl.ds(0, 1)], sem_w).wait()
                return carry

            lax.fori_loop(0, c, body, jnp.int32(0))

        def select(w, acc):
            s = pl.ds(w * 16, 16)
            node = base + w * 16 + lane
            slot = jnp.bitwise_and(w, RING - 1) * 16
            pv = pk_v[s]
            lbl = lbl_v[s]
            valid = node < N
            emask = valid & (node >= 1) & (pv >= THR * L)
            ve = plsc.load_gather(
                ering, [slot + lane, jnp.bitwise_and(lbl, 7),
                        jnp.where(valid, node - wave_i0(w), 0)])
            vu = plsc.load_gather(
                ublk, [jnp.bitwise_and(lbl, L - 1),
                       jnp.where(valid, node - ubase, 0)])
            acc = acc + jnp.where(valid, vu, 0.0)
            return acc + jnp.where(emask, ve, 0.0)

        ucopy.wait()
        for w in range(DEPTH):
            fire(jnp.int32(w))

        def wave_body(w, acc):
            pl.when(w + DEPTH < NWAVE)(lambda: fire(w + DEPTH))
            drain(w)
            return select(w, acc)

        acc = lax.fori_loop(0, NWAVE, wave_body,
                            jnp.zeros((16,), jnp.float32))
        acc_v[...] = acc
        pltpu.sync_copy(acc_v, out_hbm.at[wid])

    return k(true_labels, unary_t, edge_t, plbl_pad)


def _tc_edge_body(plbl_ref, lbl_ref, bel_ref, edge_ref, out_ref):
    g = pl.program_id(0)
    kk = lax.broadcasted_iota(jnp.int32, (1, L, N), 1)
    ii = lax.broadcasted_iota(jnp.int32, (1, L, N), 2)
    lblb = lbl_ref[...].reshape(1, 1, N)
    plblb = plbl_ref[...].reshape(1, 1, N)
    s = jnp.float32(0.0)
    for dj in range(JBLK):
        sel = ((plblb == g * JBLK + dj) & (kk == lblb) & (ii >= 1))
        s = s + jnp.sum(jnp.where(sel, edge_ref[pl.ds(dj, 1)], 0.0))

    @pl.when(g == 0)
    def _():
        bel = bel_ref[...]
        m = jnp.max(bel)
        z = m + jnp.log(jnp.sum(jnp.exp(bel - m)))
        out_ref[...] = jnp.broadcast_to(z, (1, 1))

    out_ref[...] = out_ref[...] - s


def _tc_final_body(zme_ref, part_ref, out_ref):
    out_ref[...] = zme_ref[...] - jnp.sum(part_ref[...])


def kernel(unary_potentials, edge_potentials, beliefs, parents, true_labels):
    parents = parents.astype(jnp.int32)
    labels = true_labels.astype(jnp.int32)
    unary_t = jnp.transpose(unary_potentials, (1, 0))
    edge_t = jnp.transpose(edge_potentials, (1, 2, 0))

    plbl_pad = _sc_parent_labels(parents, labels)
    partials = _sc_unary_edge_partials(labels, unary_t, edge_t, plbl_pad)
    lbl2d = labels.reshape(1, N)

    zme = pl.pallas_call(
        _tc_edge_body,
        grid=(THR // JBLK,),
        in_specs=[
            pl.BlockSpec((1, N), lambda g: (0, 0)),
            pl.BlockSpec((1, N), lambda g: (0, 0)),
            pl.BlockSpec((1, L), lambda g: (0, 0)),
            pl.BlockSpec((JBLK, L, N), lambda g: (g, 0, 0)),
        ],
        out_specs=pl.BlockSpec((1, 1), lambda g: (0, 0)),
        out_shape=jax.ShapeDtypeStruct((1, 1), jnp.float32),
    )(plbl_pad[:N].reshape(1, N), lbl2d, beliefs[0:1, :], edge_t)

    out = pl.pallas_call(
        _tc_final_body,
        out_shape=jax.ShapeDtypeStruct((1, 1), jnp.float32),
    )(zme, partials)
    return out[0, 0]


# speedup vs baseline: 1.1908x; 1.0404x over previous
import functools

import jax
import jax.numpy as jnp
from jax import lax
from jax.experimental import pallas as pl
from jax.experimental.pallas import tpu as pltpu
from jax.experimental.pallas import tpu_sc as plsc

N = 10000
L = 64
NUM_WORKERS = 32
B = 320
NP = NUM_WORKERS * B
NWAVE = B // 16
PW = 176
UW = 512
JBLK = 4
THR = 28
RING = 4
DEPTH = 3

_SC_PARAMS = pltpu.CompilerParams(needs_layout_passes=False)
_MESH = plsc.VectorSubcoreMesh(core_axis_name="c", subcore_axis_name="s")


def _sc_parent_labels(parents, true_labels):
    @functools.partial(
        pl.kernel,
        out_type=jax.ShapeDtypeStruct((NP,), jnp.int32),
        mesh=_MESH,
        scratch_types=[
            pltpu.VMEM((B,), jnp.int32),
            pltpu.VMEM((PW,), jnp.int32),
            pltpu.VMEM((B,), jnp.int32),
        ],
        compiler_params=_SC_PARAMS,
    )
    def k(parents_hbm, labels_hbm, plbl_hbm, par_v, labw_v, plbl_v):
        wid = lax.axis_index("s") * 2 + lax.axis_index("c")
        base = wid * B
        w0 = pl.multiple_of(jnp.maximum(base // 2 - 8, 0), 8)
        pltpu.sync_copy(parents_hbm.at[pl.ds(base, B)], par_v)
        pltpu.sync_copy(labels_hbm.at[pl.ds(w0, PW)], labw_v)
        for w in range(NWAVE):
            s = pl.ds(w * 16, 16)
            idx = jnp.clip(par_v[s] - w0, 0, PW - 1)
            plbl_v[s] = plsc.load_gather(labw_v, [idx])
        pltpu.sync_copy(plbl_v, plbl_hbm.at[pl.ds(base, B)])

    return k(parents, true_labels)


def _sc_unary_edge_partials(true_labels, unary_t, edge_t, plbl_pad):
    @functools.partial(
        pl.kernel,
        out_type=jax.ShapeDtypeStruct((NUM_WORKERS, 16), jnp.float32),
        mesh=_MESH,
        scratch_types=[
            pltpu.VMEM((B,), jnp.int32),
            pltpu.VMEM((B,), jnp.int32),
            pltpu.VMEM((L, UW), jnp.float32),
            pltpu.VMEM((RING * 16, 8, 128), jnp.float32),
            pltpu.VMEM((16,), jnp.float32),
            pltpu.SemaphoreType.DMA,
            pltpu.SemaphoreType.DMA,
        ],
        compiler_params=_SC_PARAMS,
    )
    def k(labels_hbm, unary_hbm, edge_hbm, plbl_hbm, out_hbm,
          lbl_v, pk_v, ublk, ering, acc_v, sem_u, sem_w):
        wid = lax.axis_index("s") * 2 + lax.axis_index("c")
        base = wid * B
        lane = lax.iota(jnp.int32, 16)
        pltpu.sync_copy(labels_hbm.at[pl.ds(base, B)], lbl_v)
        pltpu.sync_copy(plbl_hbm.at[pl.ds(base, B)], pk_v)
        ubase = pl.multiple_of(
            jnp.minimum((base // 128) * 128, 9600), 128)
        ucopy = pltpu.async_copy(
            unary_hbm.at[pl.ds(0, L), pl.ds(ubase, UW)], ublk, sem_u)

        for w in range(NWAVE):
            s = pl.ds(w * 16, 16)
            node = base + w * 16 + lane
            pk_v[s] = jnp.where(node < N, pk_v[s] * L + lbl_v[s], 0)

        def wave_i0(w):
            return jnp.minimum(((base + w * 16) // 128) * 128, 9984)

        def fire(w):
            pv = pk_v[pl.ds(w * 16, 16)]
            slot = jnp.bitwise_and(w, RING - 1) * 16
            i0 = pl.multiple_of(wave_i0(w), 128)
            for jj in range(16):
                p = jnp.sum(jnp.where(lane == jj, pv, 0))

                @pl.when(p >= THR * L)
                def _():
                    j = jnp.right_shift(p, 6)
                    l0 = pl.multiple_of(jnp.bitwise_and(p, 56), 8)
                    pltpu.async_copy(
                        edge_hbm.at[j, pl.ds(l0, 8), pl.ds(i0, 128)],
                        ering.at[slot + jj], sem_w)

        def drain(w):
            pv = pk_v[pl.ds(w * 16, 16)]
            c = jnp.sum(jnp.where(pv >= THR * L, 1, 0))

            def body(_, carry):
                pltpu.make_async_copy(
                    edge_hbm.at[pl.ds(0, 1), pl.ds(0, 8), pl.ds(0, 128)],
                    ering.at[pl.ds(0, 1)], sem_w).wait()
                return carry

            lax.fori_loop(0, c, body, jnp.int32(0))

        def select(w, acc):
            s = pl.ds(w * 16, 16)
            node = base + w * 16 + lane
            slot = jnp.bitwise_and(w, RING - 1) * 16
            pv = pk_v[s]
            lbl = lbl_v[s]
            valid = node < N
            emask = valid & (node >= 1) & (pv >= THR * L)
            ve = plsc.load_gather(
                ering, [slot + lane, jnp.bitwise_and(lbl, 7),
                        jnp.where(valid, node - wave_i0(w), 0)])
            vu = plsc.load_gather(
                ublk, [jnp.bitwise_and(lbl, L - 1),
                       jnp.where(valid, node - ubase, 0)])
            acc = acc + jnp.where(valid, vu, 0.0)
            return acc + jnp.where(emask, ve, 0.0)

        ucopy.wait()
        for w in range(DEPTH):
            fire(jnp.int32(w))

        def wave_body(w, acc):
            pl.when(w + DEPTH < NWAVE)(lambda: fire(w + DEPTH))
            drain(w)
            return select(w, acc)

        acc = lax.fori_loop(0, NWAVE, wave_body,
                            jnp.zeros((16,), jnp.float32))
        acc_v[...] = acc
        pltpu.sync_copy(acc_v, out_hbm.at[wid])

    return k(true_labels, unary_t, edge_t, plbl_pad)


def _tc_edge_body(plbl_ref, lbl_ref, bel_ref, edge_ref, out_ref):
    g = pl.program_id(0)
    kk = lax.broadcasted_iota(jnp.int32, (1, L, N), 1)
    ii = lax.broadcasted_iota(jnp.int32, (1, L, N), 2)
    lblb = lbl_ref[...].reshape(1, 1, N)
    plblb = plbl_ref[...].reshape(1, 1, N)
    s = jnp.float32(0.0)
    for dj in range(JBLK):
        sel = ((plblb == g * JBLK + dj) & (kk == lblb) & (ii >= 1))
        s = s + jnp.sum(jnp.where(sel, edge_ref[pl.ds(dj, 1)], 0.0))

    @pl.when(g == 0)
    def _():
        bel = bel_ref[...]
        m = jnp.max(bel)
        z = m + jnp.log(jnp.sum(jnp.exp(bel - m)))
        out_ref[...] = jnp.broadcast_to(z, (1, 1))

    out_ref[...] = out_ref[...] - s


def _tc_final_body(zme_ref, part_ref, out_ref):
    out_ref[...] = zme_ref[...] - jnp.sum(part_ref[...])


def kernel(unary_potentials, edge_potentials, beliefs, parents, true_labels):
    parents = parents.astype(jnp.int32)
    labels = true_labels.astype(jnp.int32)
    unary_t = jnp.transpose(unary_potentials, (1, 0))
    edge_t = jnp.transpose(edge_potentials, (1, 2, 0))

    plbl_pad = _sc_parent_labels(parents, labels)
    partials = _sc_unary_edge_partials(labels, unary_t, edge_t, plbl_pad)
    lbl2d = labels.reshape(1, N)

    zme = pl.pallas_call(
        _tc_edge_body,
        grid=(THR // JBLK,),
        in_specs=[
            pl.BlockSpec((1, N), lambda g: (0, 0)),
            pl.BlockSpec((1, N), lambda g: (0, 0)),
            pl.BlockSpec((1, L), lambda g: (0, 0)),
            pl.BlockSpec((JBLK, L, N), lambda g: (g, 0, 0)),
        ],
        out_specs=pl.BlockSpec((1, 1), lambda g: (0, 0)),
        out_shape=jax.ShapeDtypeStruct((1, 1), jnp.float32),
    )(plbl_pad[:N].reshape(1, N), lbl2d, beliefs[0:1, :], edge_t)

    out = pl.pallas_call(
        _tc_final_body,
        out_shape=jax.ShapeDtypeStruct((1, 1), jnp.float32),
    )(zme, partials)
    return out[0, 0]
